# baseline (speedup 1.0000x reference)
import jax
import jax.numpy as jnp
from jax.experimental import pallas as pl
from jax.experimental.pallas import tpu as pltpu

DIM = 512
HID = 2048
E = 8
NBUF = 3
BLK = 128
S = 2048


def _top2(logits):
    T = logits.shape[0]
    col = jax.lax.broadcasted_iota(jnp.int32, (T, E), 1)
    m1 = jnp.max(logits, axis=1, keepdims=True)
    big = jnp.int32(E)
    idx1 = jnp.min(jnp.where(logits == m1, col, big), axis=1, keepdims=True)
    masked = jnp.where(col == idx1, -jnp.inf, logits)
    m2 = jnp.max(masked, axis=1, keepdims=True)
    idx2 = jnp.min(jnp.where(masked == m2, col, big), axis=1, keepdims=True)
    e2 = jnp.exp(m2 - m1)
    p1 = 1.0 / (1.0 + e2)
    p2 = 1.0 - p1
    oh1 = (col == idx1).astype(jnp.float32)
    oh2 = (col == idx2).astype(jnp.float32)
    return oh1, oh2, p1, p2


def _moe_body(x_ref, gw_ref, w1_hbm, w2_hbm, o_ref,
              w1buf, w2buf, tok_ref, wsl_ref, nbv_ref, nbs_ref, sems, msem):
    def copy1(e, b):
        return pltpu.make_async_copy(w1_hbm.at[e], w1buf.at[b], sems.at[b, 0])

    def copy2(e, b):
        return pltpu.make_async_copy(w2_hbm.at[e], w2buf.at[b], sems.at[b, 1])

    for e in range(NBUF):
        copy1(e, e).start()
        copy2(e, e).start()

    T = x_ref.shape[0]
    xb = x_ref[...]

    logits = jax.lax.dot_general(
        xb, gw_ref[...], (((1,), (1,)), ((), ())),
        preferred_element_type=jnp.float32)
    oh1, oh2, p1, p2 = _top2(logits)
    ohc = oh1 + oh2

    counts = jnp.sum(ohc, axis=0, keepdims=True)
    nb = jax.lax.shift_right_logical(
        counts.astype(jnp.int32) + (BLK - 1), 7)
    r8 = jax.lax.broadcasted_iota(jnp.int32, (E, E), 0)
    c8 = jax.lax.broadcasted_iota(jnp.int32, (E, E), 1)
    mask8 = (r8 < c8).astype(jnp.float32)
    bs = jax.lax.dot_general(
        nb.astype(jnp.float32), mask8, (((1,), (0,)), ((), ())),
        preferred_element_type=jnp.float32)
    base_slot = bs * float(BLK)

    rT = jax.lax.broadcasted_iota(jnp.int32, (T, T), 0)
    cT = jax.lax.broadcasted_iota(jnp.int32, (T, T), 1)
    A = (cT < rT).astype(jnp.float32)
    R = jax.lax.dot_general(
        A, ohc, (((1,), (0,)), ((), ())),
        preferred_element_type=jnp.float32)

    pos1 = jnp.sum(oh1 * (base_slot + R), axis=1, keepdims=True)
    pos2 = jnp.sum(oh2 * (base_slot + R), axis=1, keepdims=True)

    siota = jax.lax.broadcasted_iota(jnp.int32, (T, S), 1)
    eqA = (pos1.astype(jnp.int32) == siota).astype(jnp.float32)
    eqB = (pos2.astype(jnp.int32) == siota).astype(jnp.float32)
    tokc = jax.lax.broadcasted_iota(jnp.int32, (T, 1), 0).astype(jnp.float32)
    onec = jnp.ones((T, 1), jnp.float32)
    ctr0 = (((0,), (0,)), ((), ()))
    tok_val = (jax.lax.dot_general(eqA, tokc, ctr0,
                                   preferred_element_type=jnp.float32)
               + jax.lax.dot_general(eqB, tokc, ctr0,
                                     preferred_element_type=jnp.float32))
    w_val = (jax.lax.dot_general(eqA, p1, ctr0,
                                 preferred_element_type=jnp.float32)
             + jax.lax.dot_general(eqB, p2, ctr0,
                                   preferred_element_type=jnp.float32))
    hit = (jax.lax.dot_general(eqA + eqB, onec, ctr0,
                               preferred_element_type=jnp.float32))
    tok_ref[...] = jnp.where(hit > 0.5, tok_val, -1.0)
    wsl_ref[...] = w_val

    pad = jnp.zeros((1, 128 - E), jnp.int32)
    nbv_ref[...] = jnp.concatenate([nb, pad], axis=1)
    meta_cp = pltpu.make_async_copy(nbv_ref, nbs_ref, msem)
    meta_cp.start()
    o_ref[...] = jnp.zeros_like(o_ref)
    meta_cp.wait()

    tiota = jax.lax.broadcasted_iota(jnp.int32, (BLK, T), 1)
    blocks_before = jnp.int32(0)
    for e in range(E):
        b = e % NBUF
        copy1(e, b).wait()
        copy2(e, b).wait()
        nb_e = nbs_ref[0, e]

        def block_body(j, carry, _b=b):
            s0 = (carry + j) * BLK
            tokv = tok_ref[pl.ds(s0, BLK), :]
            wv = wsl_ref[pl.ds(s0, BLK), :]
            sel = (tokv.astype(jnp.int32) == tiota).astype(jnp.float32)
            xs = jax.lax.dot_general(
                sel, xb, (((1,), (0,)), ((), ())),
                preferred_element_type=jnp.float32)
            hh = jax.lax.dot_general(
                xs, w1buf[_b], (((1,), (1,)), ((), ())),
                preferred_element_type=jnp.float32)
            hh = jnp.maximum(hh, 0.0)
            y = jax.lax.dot_general(
                hh, w2buf[_b], (((1,), (1,)), ((), ())),
                preferred_element_type=jnp.float32)
            y = y * wv
            o_ref[...] += jax.lax.dot_general(
                sel, y, (((0,), (0,)), ((), ())),
                preferred_element_type=jnp.float32)
            return carry

        jax.lax.fori_loop(0, nb_e, block_body, blocks_before)
        blocks_before = blocks_before + nb_e
        if e + NBUF < E:
            copy1(e + NBUF, b).start()
            copy2(e + NBUF, b).start()


@jax.jit
def kernel(x, gate_w, w1, w2):
    B, N, D = x.shape
    T = B * N
    out = pl.pallas_call(
        _moe_body,
        in_specs=[
            pl.BlockSpec(memory_space=pltpu.VMEM),
            pl.BlockSpec(memory_space=pltpu.VMEM),
            pl.BlockSpec(memory_space=pl.ANY),
            pl.BlockSpec(memory_space=pl.ANY),
        ],
        out_specs=pl.BlockSpec(memory_space=pltpu.VMEM),
        out_shape=jax.ShapeDtypeStruct((T, D), jnp.float32),
        scratch_shapes=[
            pltpu.VMEM((NBUF, HID, DIM), jnp.float32),
            pltpu.VMEM((NBUF, DIM, HID), jnp.float32),
            pltpu.VMEM((S, 1), jnp.float32),
            pltpu.VMEM((S, 1), jnp.float32),
            pltpu.VMEM((1, 128), jnp.int32),
            pltpu.SMEM((1, 128), jnp.int32),
            pltpu.SemaphoreType.DMA((NBUF, 2)),
            pltpu.SemaphoreType.DMA,
        ],
    )(x.reshape(T, D), gate_w, w1, w2)
    return out.reshape(B, N, D)
